# baseline (speedup 1.0000x reference)
import functools

import jax
import jax.numpy as jnp
from jax import lax
from jax.experimental import pallas as pl
from jax.experimental.pallas import tpu as pltpu
from jax.experimental.pallas import tpu_sc as plsc

_D = 32
_NC = 2
_NS = 16
_NW = _NC * _NS
_CH = 128
_NBUF = 4


def _make_gather(B, H):
    assert B % (_NW * _CH) == 0
    cpw = B // (_NW * _CH)
    nslab = B // _CH // H
    assert cpw % _NBUF == 0

    @functools.partial(
        pl.kernel,
        out_type=jax.ShapeDtypeStruct((H, _D // 8, nslab, 1024), jnp.float32),
        mesh=plsc.VectorSubcoreMesh(core_axis_name="c", subcore_axis_name="s"),
        compiler_params=pltpu.CompilerParams(
            use_tc_tiling_on_sc=False, needs_layout_passes=False),
        scratch_types=(
            [pltpu.VMEM((cpw, _CH), jnp.int32),
             pltpu.VMEM((_NBUF, _CH, _D), jnp.float32),
             pltpu.VMEM((_NBUF, _D * _CH), jnp.float32)]
            + [pltpu.SemaphoreType.DMA] * (2 * _NBUF)
        ),
    )
    def gather_kernel(x_hbm, tab_hbm, out_hbm, idx_v, rows_v, patch_v, *sems):
        gsems = sems[:_NBUF]
        wsems = sems[_NBUF:]
        wid = lax.axis_index("s") * _NC + lax.axis_index("c")
        rbase = wid * cpw

        pltpu.sync_copy(x_hbm.at[pl.ds(rbase, cpw)], idx_v)

        lane = lax.iota(jnp.int32, 16)
        pos = [lane * 128 + c * 2048 for c in range(2)]

        def out_tiles(j):
            gc = rbase + j
            return gc // nslab, gc % nslab

        for b in range(_NBUF):
            pltpu.async_copy(tab_hbm.at[idx_v.at[b]], rows_v.at[b], gsems[b])

        def step(g, carry):
            for b in range(_NBUF):
                j = g * _NBUF + b
                h, jb = out_tiles(j)
                pltpu.make_async_copy(
                    tab_hbm.at[idx_v.at[j]], rows_v.at[b], gsems[b]).wait()

                @pl.when(g > 0)
                def _():
                    hp, jp = out_tiles(j - _NBUF)
                    for i in range(_D // 8):
                        pltpu.make_async_copy(
                            patch_v.at[b, pl.ds(i * 1024, 1024)],
                            out_hbm.at[hp, i, jp], wsems[b]).wait()

                patch = patch_v.at[b]
                for l in range(_CH):
                    for c in range(2):
                        v = rows_v[b, l, pl.ds(c * 16, 16)]
                        plsc.store_scatter(patch, [pos[c] + l], v)

                for i in range(_D // 8):
                    pltpu.async_copy(
                        patch_v.at[b, pl.ds(i * 1024, 1024)],
                        out_hbm.at[h, i, jb], wsems[b])

                nj = j + _NBUF

                @pl.when(nj < cpw)
                def _():
                    pltpu.async_copy(
                        tab_hbm.at[idx_v.at[nj]], rows_v.at[b], gsems[b])
            return carry

        lax.fori_loop(0, cpw // _NBUF, step, 0)

        for b in range(_NBUF):
            h, jb = out_tiles(cpw - _NBUF + b)
            for i in range(_D // 8):
                pltpu.make_async_copy(
                    patch_v.at[b, pl.ds(i * 1024, 1024)],
                    out_hbm.at[h, i, jb], wsems[b]).wait()

    return gather_kernel


def kernel(x, table):
    batch, hist = x.shape
    B = batch * hist
    xf = x.astype(jnp.int32).T.reshape(B // _CH, _CH)
    out5 = _make_gather(B, hist)(xf, table)
    out6 = out5.reshape(hist, _D // 8, B // _CH // hist, 8, 128)
    return out6.transpose(2, 4, 0, 1, 3).reshape(batch, hist, _D)
